# baseline (speedup 1.0000x reference)
import dataclasses
import functools

import jax
import jax.numpy as jnp
from jax import lax
from jax.experimental import pallas as pl
from jax.experimental.pallas import tpu as pltpu
from jax.experimental.pallas import tpu_sc as plsc

F32 = jnp.float32


def _sc_compiler_params():
    cp = pltpu.CompilerParams()
    if "needs_layout_passes" in pltpu.CompilerParams.__dataclass_fields__:
        cp = dataclasses.replace(cp, needs_layout_passes=False)
    return cp


NC = 2
NS = 16
LANES = 16
NW = NC * NS


def _make_deg_kernel(N, E, T):
    assert E % NW == 0 and N % LANES == 0
    EPW = E // NW
    mesh = plsc.VectorSubcoreMesh(core_axis_name="c", subcore_axis_name="s")

    @functools.partial(
        pl.kernel,
        out_type=[jax.ShapeDtypeStruct((NW * N,), F32) for _ in range(T)],
        mesh=mesh,
        scratch_types=(
            [pltpu.VMEM((N,), F32)]
            + [pltpu.VMEM((EPW,), jnp.int32) for _ in range(2)]
            + [pltpu.SemaphoreType.DMA for _ in range(2)]
        ),
        compiler_params=_sc_compiler_params(),
    )
    def deg_kernel(*args):
        dsts = args[:T]
        outs = args[T:2 * T]
        hist = args[2 * T]
        dbuf = args[2 * T + 1:2 * T + 3]
        sem = args[2 * T + 3:2 * T + 5]
        cid = lax.axis_index("c")
        sid = lax.axis_index("s")
        wid = sid * NC + cid
        z16 = jnp.zeros((LANES,), F32)
        ones16 = jnp.ones((LANES,), F32)

        base = wid * EPW
        pltpu.async_copy(dsts[0].at[pl.ds(base, EPW)], dbuf[0], sem[0])
        for t in range(T):
            b = t % 2

            @pl.loop(0, N, step=LANES, unroll=8)
            def _(i):
                hist[pl.ds(i, LANES)] = z16

            pltpu.make_async_copy(dsts[t].at[pl.ds(base, EPW)], dbuf[b],
                                  sem[b]).wait()
            if t + 1 < T:
                pltpu.async_copy(dsts[t + 1].at[pl.ds(base, EPW)],
                                 dbuf[1 - b], sem[1 - b])

            @pl.loop(0, EPW, step=LANES, unroll=8)
            def _(i):
                idx = dbuf[b][pl.ds(i, LANES)]
                plsc.addupdate_scatter(hist, [idx], ones16)

            pltpu.sync_copy(hist, outs[t].at[pl.ds(wid * N, N)])

    return deg_kernel


def _make_scatter_kernel(N, E, H):
    assert E % NW == 0
    EPW = E // NW
    K = 40
    NB = 5
    assert EPW % K == 0 and K % 8 == 0
    NCHUNK = EPW // K
    assert NCHUNK % NB == 0 and NCHUNK >= 2 * NB
    ROWS_PS = (N // (8 * NS)) * 8
    REM = N - ROWS_PS * NS
    ZB = 48
    assert ROWS_PS % ZB == 0 and REM <= ZB and REM % 8 == 0
    mesh = plsc.VectorSubcoreMesh(core_axis_name="c", subcore_axis_name="s")

    @functools.partial(
        pl.kernel,
        out_type=jax.ShapeDtypeStruct((NC, N, H), F32),
        mesh=mesh,
        scratch_types=(
            [pltpu.VMEM((ZB, H), F32),
             pltpu.VMEM((EPW,), jnp.int32),
             pltpu.VMEM_SHARED((N, H), F32)]
            + [pltpu.VMEM((K, H), F32) for _ in range(NB)]
            + [pltpu.VMEM((K,), jnp.int32) for _ in range(NB)]
            + [pltpu.SemaphoreType.DMA for _ in range(3 * NB)]
        ),
        compiler_params=_sc_compiler_params(),
    )
    def scat_kernel(g_hbm, src_hbm, dst_hbm, out_hbm, *scr):
        zbuf, src_v, acc = scr[0], scr[1], scr[2]
        rb = scr[3:3 + NB]
        db = scr[3 + NB:3 + 2 * NB]
        gs = scr[3 + 2 * NB:3 + 3 * NB]
        ds = scr[3 + 3 * NB:3 + 4 * NB]
        ss = scr[3 + 4 * NB:3 + 5 * NB]
        cid = lax.axis_index("c")
        sid = lax.axis_index("s")
        wid = sid * NC + cid
        z16 = jnp.zeros((LANES,), F32)

        @pl.loop(0, ZB)
        def _(r):
            @pl.loop(0, H, step=LANES)
            def _(c):
                zbuf[r, pl.ds(c, LANES)] = z16

        r_base = sid * ROWS_PS

        @pl.loop(0, ROWS_PS, step=ZB)
        def _(r0):
            pltpu.sync_copy(zbuf, acc.at[pl.ds(r_base + r0, ZB)])

        @pl.when(sid == 0)
        def _():
            pltpu.sync_copy(zbuf.at[pl.ds(0, REM)],
                            acc.at[pl.ds(NS * ROWS_PS, REM)])

        base = wid * EPW

        def start_gather(j, b):
            pltpu.async_copy(g_hbm.at[src_v.at[pl.ds(j * K, K)]], rb[b],
                             gs[b])

        def wait_gather(b):
            pltpu.make_async_copy(g_hbm.at[src_v.at[pl.ds(0, K)]], rb[b],
                                  gs[b]).wait()

        def start_dst(j, b):
            pltpu.async_copy(dst_hbm.at[pl.ds(base + j * K, K)], db[b],
                             ds[b])

        def wait_dst(b):
            pltpu.make_async_copy(dst_hbm.at[pl.ds(base, K)], db[b],
                                  ds[b]).wait()

        def start_scatter(b):
            pltpu.async_copy(rb[b], acc.at[db[b]], ss[b], add=True)

        def wait_scatter(b):
            pltpu.make_async_copy(rb[b], acc.at[db[b]], ss[b]).wait()

        for b in range(NB - 1):
            start_dst(b, b)
        pltpu.sync_copy(src_hbm.at[pl.ds(base, EPW)], src_v)
        for b in range(NB - 1):
            start_gather(b, b)

        plsc.subcore_barrier()

        @pl.loop(0, NCHUNK, step=NB)
        def _(jj):
            for b in range(NB):
                pb = (b + NB - 1) % NB
                wait_gather(b)
                wait_dst(b)
                start_scatter(b)
                if b == 0:
                    @pl.when(jj > 0)
                    def _():
                        wait_scatter(pb)
                    start_dst(jj + NB - 1, pb)
                    start_gather(jj + NB - 1, pb)
                else:
                    wait_scatter(pb)

                    @pl.when(jj + b + NB - 1 < NCHUNK)
                    def _(b=b, pb=pb, jj=jj):
                        start_dst(jj + b + NB - 1, pb)
                        start_gather(jj + b + NB - 1, pb)

        wait_scatter(NB - 1)

        plsc.subcore_barrier()
        pltpu.sync_copy(acc.at[pl.ds(r_base, ROWS_PS)],
                        out_hbm.at[cid, pl.ds(r_base, ROWS_PS)])

        @pl.when(sid == 0)
        def _():
            pltpu.sync_copy(acc.at[pl.ds(NS * ROWS_PS, REM)],
                            out_hbm.at[cid, pl.ds(NS * ROWS_PS, REM)])

    return scat_kernel


def _dinv_prep(deg_p):
    N = deg_p.shape[1]

    def body(dp_ref, out_ref):
        s = jnp.sum(dp_ref[...], axis=0, keepdims=True) + 1.0
        out_ref[...] = lax.rsqrt(s).T

    return pl.pallas_call(
        body,
        out_shape=jax.ShapeDtypeStruct((N, 1), F32),
    )(deg_p)


def _stage_in(x, W_in, b_in, Wc0, dinv, R=1000):
    N, D = x.shape
    H = W_in.shape[1]
    assert N % R == 0

    def body(x_ref, win_ref, bin_ref, wc_ref, dinv_ref, g_ref):
        h0 = jnp.dot(x_ref[...], win_ref[...],
                     preferred_element_type=F32) + bin_ref[...]
        g_ref[...] = jnp.dot(h0, wc_ref[...],
                             preferred_element_type=F32) * dinv_ref[...]

    return pl.pallas_call(
        body,
        grid=(N // R,),
        in_specs=[
            pl.BlockSpec((R, D), lambda i: (i, 0)),
            pl.BlockSpec((D, H), lambda i: (0, 0)),
            pl.BlockSpec((1, H), lambda i: (0, 0)),
            pl.BlockSpec((H, H), lambda i: (0, 0)),
            pl.BlockSpec((R, 1), lambda i: (i, 0)),
        ],
        out_specs=pl.BlockSpec((R, H), lambda i: (i, 0)),
        out_shape=jax.ShapeDtypeStruct((N, H), F32),
    )(x, W_in, b_in.reshape(1, H), Wc0, dinv)


def _post_conv(p, g, dinv, bc, lng, lnb):
    s = (p[0] + p[1] + g) * dinv + bc
    m = jnp.mean(s, axis=-1, keepdims=True)
    d = s - m
    v = jnp.mean(d * d, axis=-1, keepdims=True)
    t = d * lax.rsqrt(v + 1e-5) * lng + lnb
    return jnp.maximum(t, 0.0)


def _stage_mid(part, g, hprev, dinv, bc_i, lng_i, lnb_i, Wnext, residual,
               R=1000):
    N, H = g.shape
    assert N % R == 0

    if residual:
        def body(part_ref, g_ref, hp_ref, dinv_ref, bc_ref, lng_ref, lnb_ref,
                 wn_ref, h_ref, gn_ref):
            h = _post_conv(part_ref[...], g_ref[...], dinv_ref[...],
                           bc_ref[...], lng_ref[...], lnb_ref[...])
            h = h + hp_ref[...]
            h_ref[...] = h
            gn_ref[...] = jnp.dot(h, wn_ref[...],
                                  preferred_element_type=F32) * dinv_ref[...]
        extra = [hprev]
        extra_specs = [pl.BlockSpec((R, H), lambda i: (i, 0))]
    else:
        def body(part_ref, g_ref, dinv_ref, bc_ref, lng_ref, lnb_ref,
                 wn_ref, h_ref, gn_ref):
            h = _post_conv(part_ref[...], g_ref[...], dinv_ref[...],
                           bc_ref[...], lng_ref[...], lnb_ref[...])
            h_ref[...] = h
            gn_ref[...] = jnp.dot(h, wn_ref[...],
                                  preferred_element_type=F32) * dinv_ref[...]
        extra = []
        extra_specs = []

    return pl.pallas_call(
        body,
        grid=(N // R,),
        in_specs=[
            pl.BlockSpec((2, R, H), lambda i: (0, i, 0)),
            pl.BlockSpec((R, H), lambda i: (i, 0)),
            *extra_specs,
            pl.BlockSpec((R, 1), lambda i: (i, 0)),
            pl.BlockSpec((1, H), lambda i: (0, 0)),
            pl.BlockSpec((1, H), lambda i: (0, 0)),
            pl.BlockSpec((1, H), lambda i: (0, 0)),
            pl.BlockSpec((H, H), lambda i: (0, 0)),
        ],
        out_specs=[
            pl.BlockSpec((R, H), lambda i: (i, 0)),
            pl.BlockSpec((R, H), lambda i: (i, 0)),
        ],
        out_shape=[
            jax.ShapeDtypeStruct((N, H), F32),
            jax.ShapeDtypeStruct((N, H), F32),
        ],
    )(part, g, *extra, dinv, bc_i.reshape(1, H), lng_i.reshape(1, H),
      lnb_i.reshape(1, H), Wnext)


def _stage_last(part, g, hprev, dinv, bc_i, lng_i, lnb_i, R=1000):
    N, H = g.shape
    assert N % R == 0

    def body(part_ref, g_ref, hp_ref, dinv_ref, bc_ref, lng_ref, lnb_ref,
             out_ref):
        h = _post_conv(part_ref[...], g_ref[...], dinv_ref[...],
                       bc_ref[...], lng_ref[...], lnb_ref[...])
        h = h + hp_ref[...]

        @pl.when(pl.program_id(0) == 0)
        def _():
            out_ref[...] = jnp.zeros_like(out_ref)

        out_ref[...] += jnp.sum(h, axis=0, keepdims=True)

    return pl.pallas_call(
        body,
        grid=(N // R,),
        in_specs=[
            pl.BlockSpec((2, R, H), lambda i: (0, i, 0)),
            pl.BlockSpec((R, H), lambda i: (i, 0)),
            pl.BlockSpec((R, H), lambda i: (i, 0)),
            pl.BlockSpec((R, 1), lambda i: (i, 0)),
            pl.BlockSpec((1, H), lambda i: (0, 0)),
            pl.BlockSpec((1, H), lambda i: (0, 0)),
            pl.BlockSpec((1, H), lambda i: (0, 0)),
        ],
        out_specs=pl.BlockSpec((1, H), lambda i: (0, 0)),
        out_shape=jax.ShapeDtypeStruct((1, H), F32),
    )(part, g, hprev, dinv, bc_i.reshape(1, H), lng_i.reshape(1, H),
      lnb_i.reshape(1, H))


def _last_and_head(part, g, hprev, dinv, bc_i, lng_i, lnb_i, prev_sums,
                   Wih_t, Whh_t, bih, bhh, W1, b1, W2r, b2, R=1000):
    N, H = g.shape
    assert N % R == 0
    NG = N // R
    T = prev_sums.shape[0] + 1
    Hmid = W1.shape[1]

    def body(part_ref, g_ref, hp_ref, dinv_ref, bc_ref, lng_ref, lnb_ref,
             prev_ref, wih_ref, whh_ref, bih_ref, bhh_ref, w1_ref, b1_ref,
             w2_ref, b2_ref, pred_ref, final_ref, acc_ref):
        h3 = _post_conv(part_ref[...], g_ref[...], dinv_ref[...],
                        bc_ref[...], lng_ref[...], lnb_ref[...])
        h3 = h3 + hp_ref[...]

        @pl.when(pl.program_id(0) == 0)
        def _():
            acc_ref[...] = jnp.zeros_like(acc_ref)

        acc_ref[...] += jnp.sum(h3, axis=0, keepdims=True)

        @pl.when(pl.program_id(0) == NG - 1)
        def _():
            _head_compute(prev_ref, acc_ref, wih_ref, whh_ref, bih_ref,
                          bhh_ref, w1_ref, b1_ref, w2_ref, b2_ref,
                          pred_ref, final_ref, T, H, N)

    return pl.pallas_call(
        body,
        grid=(NG,),
        in_specs=[
            pl.BlockSpec((2, R, H), lambda i: (0, i, 0)),
            pl.BlockSpec((R, H), lambda i: (i, 0)),
            pl.BlockSpec((R, H), lambda i: (i, 0)),
            pl.BlockSpec((R, 1), lambda i: (i, 0)),
            pl.BlockSpec((1, H), lambda i: (0, 0)),
            pl.BlockSpec((1, H), lambda i: (0, 0)),
            pl.BlockSpec((1, H), lambda i: (0, 0)),
            pl.BlockSpec(prev_sums.shape, lambda i: (0, 0)),
            pl.BlockSpec(Wih_t.shape, lambda i: (0, 0, 0)),
            pl.BlockSpec(Whh_t.shape, lambda i: (0, 0, 0)),
            pl.BlockSpec(bih.shape, lambda i: (0, 0)),
            pl.BlockSpec(bhh.shape, lambda i: (0, 0)),
            pl.BlockSpec(W1.shape, lambda i: (0, 0)),
            pl.BlockSpec((1, Hmid), lambda i: (0, 0)),
            pl.BlockSpec(W2r.shape, lambda i: (0, 0)),
            pl.BlockSpec((1, 1), lambda i: (0, 0)),
        ],
        out_specs=[
            pl.BlockSpec((1, 1), lambda i: (0, 0)),
            pl.BlockSpec((1, H), lambda i: (0, 0)),
        ],
        out_shape=[
            jax.ShapeDtypeStruct((1, 1), F32),
            jax.ShapeDtypeStruct((1, H), F32),
        ],
        scratch_shapes=[pltpu.VMEM((1, H), F32)],
    )(part, g, hprev, dinv, bc_i.reshape(1, H), lng_i.reshape(1, H),
      lnb_i.reshape(1, H), prev_sums, Wih_t, Whh_t, bih, bhh, W1,
      b1.reshape(1, Hmid), W2r, b2.reshape(1, 1))


def _head_compute(prev_ref, acc_ref, wih_ref, whh_ref, bih_ref, bhh_ref,
                  w1_ref, b1_ref, w2_ref, b2_ref, pred_ref, final_ref,
                  T, H, N):
        inv_n = F32(1.0 / N)
        xs = [prev_ref[pl.ds(t, 1), :] * inv_n for t in range(T - 1)]
        xs.append(acc_ref[...] * inv_n)
        for l in range(2):
            wih = wih_ref[l]
            whh = whh_ref[l]
            b = bih_ref[pl.ds(l, 1), :] + bhh_ref[pl.ds(l, 1), :]
            h = jnp.zeros((1, H), F32)
            c = jnp.zeros((1, H), F32)
            ys = []
            for t in range(T):
                z = (jnp.dot(xs[t], wih, preferred_element_type=F32)
                     + jnp.dot(h, whh, preferred_element_type=F32) + b)
                zi = z[:, 0:H]
                zf = z[:, H:2 * H]
                zg = z[:, 2 * H:3 * H]
                zo = z[:, 3 * H:4 * H]
                c = jax.nn.sigmoid(zf) * c + jax.nn.sigmoid(zi) * jnp.tanh(zg)
                h = jax.nn.sigmoid(zo) * jnp.tanh(c)
                ys.append(h)
            xs = ys
        final = xs[-1]
        hmid = jnp.maximum(
            jnp.dot(final, w1_ref[...], preferred_element_type=F32)
            + b1_ref[...], 0.0)
        pred_ref[...] = (jnp.sum(hmid * w2_ref[...], axis=-1, keepdims=True)
                         + b2_ref[...])
        final_ref[...] = final


@jax.jit
def kernel(x_0, x_1, x_2, x_3,
           edge_index_0, edge_index_1, edge_index_2, edge_index_3,
           W_in, b_in, Wc, bc, ln_g, ln_b,
           lstm_Wih, lstm_Whh, lstm_bih, lstm_bhh, W1, b1, W2, b2):
    xs = [x_0, x_1, x_2, x_3]
    eis = [edge_index_0, edge_index_1, edge_index_2, edge_index_3]
    N, D = x_0.shape
    E = edge_index_0.shape[1]
    H = W_in.shape[1]
    T = len(xs)

    deg_kernel = _make_deg_kernel(N, E, T)
    scat_kernel = _make_scatter_kernel(N, E, H)

    deg_ps = deg_kernel(*[ei[1] for ei in eis])

    emb_sums = []
    for t in range(T):
        src = eis[t][0]
        dst = eis[t][1]
        dinv = _dinv_prep(deg_ps[t].reshape(NW, N))
        g0 = _stage_in(xs[t], W_in, b_in, Wc[0], dinv)
        part0 = scat_kernel(g0, src, dst)
        h1, g1 = _stage_mid(part0, g0, None, dinv, bc[0], ln_g[0], ln_b[0],
                            Wc[1], residual=False)
        part1 = scat_kernel(g1, src, dst)
        h2, g2 = _stage_mid(part1, g1, h1, dinv, bc[1], ln_g[1], ln_b[1],
                            Wc[2], residual=True)
        part2 = scat_kernel(g2, src, dst)
        if t < T - 1:
            emb_sums.append(_stage_last(part2, g2, h2, dinv, bc[2], ln_g[2],
                                        ln_b[2]))
        else:
            last_args = (part2, g2, h2, dinv)

    prev_sums = jnp.concatenate(emb_sums, axis=0)
    Wih_t = jnp.swapaxes(lstm_Wih, 1, 2)
    Whh_t = jnp.swapaxes(lstm_Whh, 1, 2)
    pred, final = _last_and_head(*last_args, bc[2], ln_g[2], ln_b[2],
                                 prev_sums, Wih_t, Whh_t, lstm_bih,
                                 lstm_bhh, W1, b1, W2.reshape(1, -1), b2)
    return pred, final


# speedup vs baseline: 1.0071x; 1.0071x over previous
import dataclasses
import functools

import jax
import jax.numpy as jnp
from jax import lax
from jax.experimental import pallas as pl
from jax.experimental.pallas import tpu as pltpu
from jax.experimental.pallas import tpu_sc as plsc

F32 = jnp.float32


def _sc_compiler_params():
    cp = pltpu.CompilerParams()
    if "needs_layout_passes" in pltpu.CompilerParams.__dataclass_fields__:
        cp = dataclasses.replace(cp, needs_layout_passes=False)
    return cp


NC = 2
NS = 16
LANES = 16
NW = NC * NS


def _make_deg_kernel(N, E, T, NH=4):
    assert E % NW == 0 and N % LANES == 0
    EPW = E // NW
    MAIN = (EPW // (NH * LANES)) * (NH * LANES)
    assert (EPW - MAIN) % LANES == 0
    mesh = plsc.VectorSubcoreMesh(core_axis_name="c", subcore_axis_name="s")

    @functools.partial(
        pl.kernel,
        out_type=[jax.ShapeDtypeStruct((NH * NW * N,), F32)
                  for _ in range(T)],
        mesh=mesh,
        scratch_types=(
            [pltpu.VMEM((N,), F32) for _ in range(NH)]
            + [pltpu.VMEM((EPW,), jnp.int32) for _ in range(2)]
            + [pltpu.SemaphoreType.DMA for _ in range(2)]
        ),
        compiler_params=_sc_compiler_params(),
    )
    def deg_kernel(*args):
        dsts = args[:T]
        outs = args[T:2 * T]
        hists = args[2 * T:2 * T + NH]
        dbuf = args[2 * T + NH:2 * T + NH + 2]
        sem = args[2 * T + NH + 2:2 * T + NH + 4]
        cid = lax.axis_index("c")
        sid = lax.axis_index("s")
        wid = sid * NC + cid
        z16 = jnp.zeros((LANES,), F32)
        ones16 = jnp.ones((LANES,), F32)

        base = wid * EPW
        pltpu.async_copy(dsts[0].at[pl.ds(base, EPW)], dbuf[0], sem[0])
        for t in range(T):
            b = t % 2
            for hh in range(NH):
                @pl.loop(0, N, step=LANES, unroll=8)
                def _(i, hh=hh):
                    hists[hh][pl.ds(i, LANES)] = z16

            pltpu.make_async_copy(dsts[t].at[pl.ds(base, EPW)], dbuf[b],
                                  sem[b]).wait()
            if t + 1 < T:
                pltpu.async_copy(dsts[t + 1].at[pl.ds(base, EPW)],
                                 dbuf[1 - b], sem[1 - b])

            @pl.loop(0, MAIN, step=NH * LANES, unroll=2)
            def _(i, b=b):
                for hh in range(NH):
                    idx = dbuf[b][pl.ds(i + hh * LANES, LANES)]
                    plsc.addupdate_scatter(hists[hh], [idx], ones16)

            @pl.loop(MAIN, EPW, step=LANES)
            def _(i, b=b):
                idx = dbuf[b][pl.ds(i, LANES)]
                plsc.addupdate_scatter(hists[0], [idx], ones16)

            for hh in range(NH):
                pltpu.sync_copy(hists[hh],
                                outs[t].at[pl.ds((hh * NW + wid) * N, N)])

    return deg_kernel


def _make_scatter_kernel(N, E, H):
    assert E % NW == 0
    EPW = E // NW
    K = 40
    NB = 5
    assert EPW % K == 0 and K % 8 == 0
    NCHUNK = EPW // K
    assert NCHUNK % NB == 0 and NCHUNK >= 2 * NB
    ROWS_PS = (N // (8 * NS)) * 8
    REM = N - ROWS_PS * NS
    ZB = 48
    assert ROWS_PS % ZB == 0 and REM <= ZB and REM % 8 == 0
    mesh = plsc.VectorSubcoreMesh(core_axis_name="c", subcore_axis_name="s")

    @functools.partial(
        pl.kernel,
        out_type=jax.ShapeDtypeStruct((NC, N, H), F32),
        mesh=mesh,
        scratch_types=(
            [pltpu.VMEM((ZB, H), F32),
             pltpu.VMEM((EPW,), jnp.int32),
             pltpu.VMEM_SHARED((N, H), F32)]
            + [pltpu.VMEM((K, H), F32) for _ in range(NB)]
            + [pltpu.VMEM((K,), jnp.int32) for _ in range(NB)]
            + [pltpu.SemaphoreType.DMA for _ in range(3 * NB)]
        ),
        compiler_params=_sc_compiler_params(),
    )
    def scat_kernel(g_hbm, src_hbm, dst_hbm, out_hbm, *scr):
        zbuf, src_v, acc = scr[0], scr[1], scr[2]
        rb = scr[3:3 + NB]
        db = scr[3 + NB:3 + 2 * NB]
        gs = scr[3 + 2 * NB:3 + 3 * NB]
        ds = scr[3 + 3 * NB:3 + 4 * NB]
        ss = scr[3 + 4 * NB:3 + 5 * NB]
        cid = lax.axis_index("c")
        sid = lax.axis_index("s")
        wid = sid * NC + cid
        z16 = jnp.zeros((LANES,), F32)

        @pl.loop(0, ZB)
        def _(r):
            @pl.loop(0, H, step=LANES)
            def _(c):
                zbuf[r, pl.ds(c, LANES)] = z16

        r_base = sid * ROWS_PS

        @pl.loop(0, ROWS_PS, step=ZB)
        def _(r0):
            pltpu.sync_copy(zbuf, acc.at[pl.ds(r_base + r0, ZB)])

        @pl.when(sid == 0)
        def _():
            pltpu.sync_copy(zbuf.at[pl.ds(0, REM)],
                            acc.at[pl.ds(NS * ROWS_PS, REM)])

        base = wid * EPW

        def start_gather(j, b):
            pltpu.async_copy(g_hbm.at[src_v.at[pl.ds(j * K, K)]], rb[b],
                             gs[b])

        def wait_gather(b):
            pltpu.make_async_copy(g_hbm.at[src_v.at[pl.ds(0, K)]], rb[b],
                                  gs[b]).wait()

        def start_dst(j, b):
            pltpu.async_copy(dst_hbm.at[pl.ds(base + j * K, K)], db[b],
                             ds[b])

        def wait_dst(b):
            pltpu.make_async_copy(dst_hbm.at[pl.ds(base, K)], db[b],
                                  ds[b]).wait()

        def start_scatter(b):
            pltpu.async_copy(rb[b], acc.at[db[b]], ss[b], add=True)

        def wait_scatter(b):
            pltpu.make_async_copy(rb[b], acc.at[db[b]], ss[b]).wait()

        for b in range(NB - 1):
            start_dst(b, b)
        pltpu.sync_copy(src_hbm.at[pl.ds(base, EPW)], src_v)
        for b in range(NB - 1):
            start_gather(b, b)

        plsc.subcore_barrier()

        @pl.loop(0, NCHUNK, step=NB)
        def _(jj):
            for b in range(NB):
                pb = (b + NB - 1) % NB
                wait_gather(b)
                wait_dst(b)
                start_scatter(b)
                if b == 0:
                    @pl.when(jj > 0)
                    def _():
                        wait_scatter(pb)
                    start_dst(jj + NB - 1, pb)
                    start_gather(jj + NB - 1, pb)
                else:
                    wait_scatter(pb)

                    @pl.when(jj + b + NB - 1 < NCHUNK)
                    def _(b=b, pb=pb, jj=jj):
                        start_dst(jj + b + NB - 1, pb)
                        start_gather(jj + b + NB - 1, pb)

        wait_scatter(NB - 1)

        plsc.subcore_barrier()
        pltpu.sync_copy(acc.at[pl.ds(r_base, ROWS_PS)],
                        out_hbm.at[cid, pl.ds(r_base, ROWS_PS)])

        @pl.when(sid == 0)
        def _():
            pltpu.sync_copy(acc.at[pl.ds(NS * ROWS_PS, REM)],
                            out_hbm.at[cid, pl.ds(NS * ROWS_PS, REM)])

    return scat_kernel


def _dinv_prep(deg_p):
    N = deg_p.shape[1]

    def body(dp_ref, out_ref):
        s = jnp.sum(dp_ref[...], axis=0, keepdims=True) + 1.0
        out_ref[...] = lax.rsqrt(s).T

    return pl.pallas_call(
        body,
        out_shape=jax.ShapeDtypeStruct((N, 1), F32),
    )(deg_p)


def _stage_in(x, W_in, b_in, Wc0, dinv, R=1000):
    N, D = x.shape
    H = W_in.shape[1]
    assert N % R == 0

    def body(x_ref, win_ref, bin_ref, wc_ref, dinv_ref, g_ref):
        h0 = jnp.dot(x_ref[...], win_ref[...],
                     preferred_element_type=F32) + bin_ref[...]
        g_ref[...] = jnp.dot(h0, wc_ref[...],
                             preferred_element_type=F32) * dinv_ref[...]

    return pl.pallas_call(
        body,
        grid=(N // R,),
        in_specs=[
            pl.BlockSpec((R, D), lambda i: (i, 0)),
            pl.BlockSpec((D, H), lambda i: (0, 0)),
            pl.BlockSpec((1, H), lambda i: (0, 0)),
            pl.BlockSpec((H, H), lambda i: (0, 0)),
            pl.BlockSpec((R, 1), lambda i: (i, 0)),
        ],
        out_specs=pl.BlockSpec((R, H), lambda i: (i, 0)),
        out_shape=jax.ShapeDtypeStruct((N, H), F32),
    )(x, W_in, b_in.reshape(1, H), Wc0, dinv)


def _post_conv(p, g, dinv, bc, lng, lnb):
    s = (p[0] + p[1] + g) * dinv + bc
    m = jnp.mean(s, axis=-1, keepdims=True)
    d = s - m
    v = jnp.mean(d * d, axis=-1, keepdims=True)
    t = d * lax.rsqrt(v + 1e-5) * lng + lnb
    return jnp.maximum(t, 0.0)


def _stage_mid(part, g, hprev, dinv, bc_i, lng_i, lnb_i, Wnext, residual,
               R=1000):
    N, H = g.shape
    assert N % R == 0

    if residual:
        def body(part_ref, g_ref, hp_ref, dinv_ref, bc_ref, lng_ref, lnb_ref,
                 wn_ref, h_ref, gn_ref):
            h = _post_conv(part_ref[...], g_ref[...], dinv_ref[...],
                           bc_ref[...], lng_ref[...], lnb_ref[...])
            h = h + hp_ref[...]
            h_ref[...] = h
            gn_ref[...] = jnp.dot(h, wn_ref[...],
                                  preferred_element_type=F32) * dinv_ref[...]
        extra = [hprev]
        extra_specs = [pl.BlockSpec((R, H), lambda i: (i, 0))]
    else:
        def body(part_ref, g_ref, dinv_ref, bc_ref, lng_ref, lnb_ref,
                 wn_ref, h_ref, gn_ref):
            h = _post_conv(part_ref[...], g_ref[...], dinv_ref[...],
                           bc_ref[...], lng_ref[...], lnb_ref[...])
            h_ref[...] = h
            gn_ref[...] = jnp.dot(h, wn_ref[...],
                                  preferred_element_type=F32) * dinv_ref[...]
        extra = []
        extra_specs = []

    return pl.pallas_call(
        body,
        grid=(N // R,),
        in_specs=[
            pl.BlockSpec((2, R, H), lambda i: (0, i, 0)),
            pl.BlockSpec((R, H), lambda i: (i, 0)),
            *extra_specs,
            pl.BlockSpec((R, 1), lambda i: (i, 0)),
            pl.BlockSpec((1, H), lambda i: (0, 0)),
            pl.BlockSpec((1, H), lambda i: (0, 0)),
            pl.BlockSpec((1, H), lambda i: (0, 0)),
            pl.BlockSpec((H, H), lambda i: (0, 0)),
        ],
        out_specs=[
            pl.BlockSpec((R, H), lambda i: (i, 0)),
            pl.BlockSpec((R, H), lambda i: (i, 0)),
        ],
        out_shape=[
            jax.ShapeDtypeStruct((N, H), F32),
            jax.ShapeDtypeStruct((N, H), F32),
        ],
    )(part, g, *extra, dinv, bc_i.reshape(1, H), lng_i.reshape(1, H),
      lnb_i.reshape(1, H), Wnext)


def _stage_last(part, g, hprev, dinv, bc_i, lng_i, lnb_i, R=1000):
    N, H = g.shape
    assert N % R == 0

    def body(part_ref, g_ref, hp_ref, dinv_ref, bc_ref, lng_ref, lnb_ref,
             out_ref):
        h = _post_conv(part_ref[...], g_ref[...], dinv_ref[...],
                       bc_ref[...], lng_ref[...], lnb_ref[...])
        h = h + hp_ref[...]

        @pl.when(pl.program_id(0) == 0)
        def _():
            out_ref[...] = jnp.zeros_like(out_ref)

        out_ref[...] += jnp.sum(h, axis=0, keepdims=True)

    return pl.pallas_call(
        body,
        grid=(N // R,),
        in_specs=[
            pl.BlockSpec((2, R, H), lambda i: (0, i, 0)),
            pl.BlockSpec((R, H), lambda i: (i, 0)),
            pl.BlockSpec((R, H), lambda i: (i, 0)),
            pl.BlockSpec((R, 1), lambda i: (i, 0)),
            pl.BlockSpec((1, H), lambda i: (0, 0)),
            pl.BlockSpec((1, H), lambda i: (0, 0)),
            pl.BlockSpec((1, H), lambda i: (0, 0)),
        ],
        out_specs=pl.BlockSpec((1, H), lambda i: (0, 0)),
        out_shape=jax.ShapeDtypeStruct((1, H), F32),
    )(part, g, hprev, dinv, bc_i.reshape(1, H), lng_i.reshape(1, H),
      lnb_i.reshape(1, H))


def _last_and_head(part, g, hprev, dinv, bc_i, lng_i, lnb_i, prev_sums,
                   Wih_t, Whh_t, bih, bhh, W1, b1, W2r, b2, R=1000):
    N, H = g.shape
    assert N % R == 0
    NG = N // R
    T = prev_sums.shape[0] + 1
    Hmid = W1.shape[1]

    def body(part_ref, g_ref, hp_ref, dinv_ref, bc_ref, lng_ref, lnb_ref,
             prev_ref, wih_ref, whh_ref, bih_ref, bhh_ref, w1_ref, b1_ref,
             w2_ref, b2_ref, pred_ref, final_ref, acc_ref):
        h3 = _post_conv(part_ref[...], g_ref[...], dinv_ref[...],
                        bc_ref[...], lng_ref[...], lnb_ref[...])
        h3 = h3 + hp_ref[...]

        @pl.when(pl.program_id(0) == 0)
        def _():
            acc_ref[...] = jnp.zeros_like(acc_ref)

        acc_ref[...] += jnp.sum(h3, axis=0, keepdims=True)

        @pl.when(pl.program_id(0) == NG - 1)
        def _():
            _head_compute(prev_ref, acc_ref, wih_ref, whh_ref, bih_ref,
                          bhh_ref, w1_ref, b1_ref, w2_ref, b2_ref,
                          pred_ref, final_ref, T, H, N)

    return pl.pallas_call(
        body,
        grid=(NG,),
        in_specs=[
            pl.BlockSpec((2, R, H), lambda i: (0, i, 0)),
            pl.BlockSpec((R, H), lambda i: (i, 0)),
            pl.BlockSpec((R, H), lambda i: (i, 0)),
            pl.BlockSpec((R, 1), lambda i: (i, 0)),
            pl.BlockSpec((1, H), lambda i: (0, 0)),
            pl.BlockSpec((1, H), lambda i: (0, 0)),
            pl.BlockSpec((1, H), lambda i: (0, 0)),
            pl.BlockSpec(prev_sums.shape, lambda i: (0, 0)),
            pl.BlockSpec(Wih_t.shape, lambda i: (0, 0, 0)),
            pl.BlockSpec(Whh_t.shape, lambda i: (0, 0, 0)),
            pl.BlockSpec(bih.shape, lambda i: (0, 0)),
            pl.BlockSpec(bhh.shape, lambda i: (0, 0)),
            pl.BlockSpec(W1.shape, lambda i: (0, 0)),
            pl.BlockSpec((1, Hmid), lambda i: (0, 0)),
            pl.BlockSpec(W2r.shape, lambda i: (0, 0)),
            pl.BlockSpec((1, 1), lambda i: (0, 0)),
        ],
        out_specs=[
            pl.BlockSpec((1, 1), lambda i: (0, 0)),
            pl.BlockSpec((1, H), lambda i: (0, 0)),
        ],
        out_shape=[
            jax.ShapeDtypeStruct((1, 1), F32),
            jax.ShapeDtypeStruct((1, H), F32),
        ],
        scratch_shapes=[pltpu.VMEM((1, H), F32)],
    )(part, g, hprev, dinv, bc_i.reshape(1, H), lng_i.reshape(1, H),
      lnb_i.reshape(1, H), prev_sums, Wih_t, Whh_t, bih, bhh, W1,
      b1.reshape(1, Hmid), W2r, b2.reshape(1, 1))


def _head_compute(prev_ref, acc_ref, wih_ref, whh_ref, bih_ref, bhh_ref,
                  w1_ref, b1_ref, w2_ref, b2_ref, pred_ref, final_ref,
                  T, H, N):
        inv_n = F32(1.0 / N)
        xs = [prev_ref[pl.ds(t, 1), :] * inv_n for t in range(T - 1)]
        xs.append(acc_ref[...] * inv_n)
        for l in range(2):
            wih = wih_ref[l]
            whh = whh_ref[l]
            b = bih_ref[pl.ds(l, 1), :] + bhh_ref[pl.ds(l, 1), :]
            h = jnp.zeros((1, H), F32)
            c = jnp.zeros((1, H), F32)
            ys = []
            for t in range(T):
                z = (jnp.dot(xs[t], wih, preferred_element_type=F32)
                     + jnp.dot(h, whh, preferred_element_type=F32) + b)
                zi = z[:, 0:H]
                zf = z[:, H:2 * H]
                zg = z[:, 2 * H:3 * H]
                zo = z[:, 3 * H:4 * H]
                c = jax.nn.sigmoid(zf) * c + jax.nn.sigmoid(zi) * jnp.tanh(zg)
                h = jax.nn.sigmoid(zo) * jnp.tanh(c)
                ys.append(h)
            xs = ys
        final = xs[-1]
        hmid = jnp.maximum(
            jnp.dot(final, w1_ref[...], preferred_element_type=F32)
            + b1_ref[...], 0.0)
        pred_ref[...] = (jnp.sum(hmid * w2_ref[...], axis=-1, keepdims=True)
                         + b2_ref[...])
        final_ref[...] = final


@jax.jit
def kernel(x_0, x_1, x_2, x_3,
           edge_index_0, edge_index_1, edge_index_2, edge_index_3,
           W_in, b_in, Wc, bc, ln_g, ln_b,
           lstm_Wih, lstm_Whh, lstm_bih, lstm_bhh, W1, b1, W2, b2):
    xs = [x_0, x_1, x_2, x_3]
    eis = [edge_index_0, edge_index_1, edge_index_2, edge_index_3]
    N, D = x_0.shape
    E = edge_index_0.shape[1]
    H = W_in.shape[1]
    T = len(xs)

    deg_kernel0 = _make_deg_kernel(N, E, 1)
    deg_kernel_rest = _make_deg_kernel(N, E, T - 1)
    scat_kernel = _make_scatter_kernel(N, E, H)

    deg_p0 = deg_kernel0(eis[0][1])
    if isinstance(deg_p0, (list, tuple)):
        deg_p0 = deg_p0[0]
    deg_rest = deg_kernel_rest(*[ei[1] for ei in eis[1:]])
    deg_ps = [deg_p0] + list(deg_rest)

    emb_sums = []
    for t in range(T):
        src = eis[t][0]
        dst = eis[t][1]
        dinv = _dinv_prep(deg_ps[t].reshape(-1, N))
        g0 = _stage_in(xs[t], W_in, b_in, Wc[0], dinv)
        part0 = scat_kernel(g0, src, dst)
        h1, g1 = _stage_mid(part0, g0, None, dinv, bc[0], ln_g[0], ln_b[0],
                            Wc[1], residual=False)
        part1 = scat_kernel(g1, src, dst)
        h2, g2 = _stage_mid(part1, g1, h1, dinv, bc[1], ln_g[1], ln_b[1],
                            Wc[2], residual=True)
        part2 = scat_kernel(g2, src, dst)
        if t < T - 1:
            emb_sums.append(_stage_last(part2, g2, h2, dinv, bc[2], ln_g[2],
                                        ln_b[2]))
        else:
            last_args = (part2, g2, h2, dinv)

    prev_sums = jnp.concatenate(emb_sums, axis=0)
    Wih_t = jnp.swapaxes(lstm_Wih, 1, 2)
    Whh_t = jnp.swapaxes(lstm_Whh, 1, 2)
    pred, final = _last_and_head(*last_args, bc[2], ln_g[2], ln_b[2],
                                 prev_sums, Wih_t, Whh_t, lstm_bih,
                                 lstm_bhh, W1, b1, W2.reshape(1, -1), b2)
    return pred, final


# speedup vs baseline: 1.0172x; 1.0100x over previous
import dataclasses
import functools

import jax
import jax.numpy as jnp
from jax import lax
from jax.experimental import pallas as pl
from jax.experimental.pallas import tpu as pltpu
from jax.experimental.pallas import tpu_sc as plsc

F32 = jnp.float32


def _sc_compiler_params():
    cp = pltpu.CompilerParams()
    if "needs_layout_passes" in pltpu.CompilerParams.__dataclass_fields__:
        cp = dataclasses.replace(cp, needs_layout_passes=False)
    return cp


NC = 2
NS = 16
LANES = 16
NW = NC * NS


def _make_deg_kernel(N, E, T):
    assert E % NW == 0 and N % LANES == 0
    EPW = E // NW
    mesh = plsc.VectorSubcoreMesh(core_axis_name="c", subcore_axis_name="s")

    @functools.partial(
        pl.kernel,
        out_type=[jax.ShapeDtypeStruct((NW * N,), F32) for _ in range(T)],
        mesh=mesh,
        scratch_types=(
            [pltpu.VMEM((N,), F32)]
            + [pltpu.VMEM((EPW,), jnp.int32) for _ in range(2)]
            + [pltpu.SemaphoreType.DMA for _ in range(2)]
        ),
        compiler_params=_sc_compiler_params(),
    )
    def deg_kernel(*args):
        dsts = args[:T]
        outs = args[T:2 * T]
        hist = args[2 * T]
        dbuf = args[2 * T + 1:2 * T + 3]
        sem = args[2 * T + 3:2 * T + 5]
        cid = lax.axis_index("c")
        sid = lax.axis_index("s")
        wid = sid * NC + cid
        z16 = jnp.zeros((LANES,), F32)
        ones16 = jnp.ones((LANES,), F32)

        base = wid * EPW
        pltpu.async_copy(dsts[0].at[pl.ds(base, EPW)], dbuf[0], sem[0])
        for t in range(T):
            b = t % 2

            @pl.loop(0, N, step=LANES, unroll=8)
            def _(i):
                hist[pl.ds(i, LANES)] = z16

            pltpu.make_async_copy(dsts[t].at[pl.ds(base, EPW)], dbuf[b],
                                  sem[b]).wait()
            if t + 1 < T:
                pltpu.async_copy(dsts[t + 1].at[pl.ds(base, EPW)],
                                 dbuf[1 - b], sem[1 - b])

            @pl.loop(0, EPW, step=LANES, unroll=8)
            def _(i):
                idx = dbuf[b][pl.ds(i, LANES)]
                plsc.addupdate_scatter(hist, [idx], ones16)

            pltpu.sync_copy(hist, outs[t].at[pl.ds(wid * N, N)])

    return deg_kernel


def _make_scatter_kernel(N, E, H):
    assert E % NW == 0
    EPW = E // NW
    K = 40
    NB = 5
    assert EPW % K == 0 and K % 8 == 0
    NCHUNK = EPW // K
    assert NCHUNK % NB == 0 and NCHUNK >= 2 * NB
    ROWS_PS = (N // (8 * NS)) * 8
    REM = N - ROWS_PS * NS
    ZB = 48
    assert ROWS_PS % ZB == 0 and REM <= ZB and REM % 8 == 0
    mesh = plsc.VectorSubcoreMesh(core_axis_name="c", subcore_axis_name="s")

    @functools.partial(
        pl.kernel,
        out_type=jax.ShapeDtypeStruct((NC, N, H), F32),
        mesh=mesh,
        scratch_types=(
            [pltpu.VMEM((ZB, H), F32),
             pltpu.VMEM((EPW,), jnp.int32),
             pltpu.VMEM_SHARED((N, H), F32)]
            + [pltpu.VMEM((K, H), F32) for _ in range(NB)]
            + [pltpu.VMEM((K,), jnp.int32) for _ in range(NB)]
            + [pltpu.SemaphoreType.DMA for _ in range(3 * NB)]
        ),
        compiler_params=_sc_compiler_params(),
    )
    def scat_kernel(g_hbm, src_hbm, dst_hbm, out_hbm, *scr):
        zbuf, src_v, acc = scr[0], scr[1], scr[2]
        rb = scr[3:3 + NB]
        db = scr[3 + NB:3 + 2 * NB]
        gs = scr[3 + 2 * NB:3 + 3 * NB]
        ds = scr[3 + 3 * NB:3 + 4 * NB]
        ss = scr[3 + 4 * NB:3 + 5 * NB]
        cid = lax.axis_index("c")
        sid = lax.axis_index("s")
        wid = sid * NC + cid
        z16 = jnp.zeros((LANES,), F32)

        @pl.loop(0, ZB)
        def _(r):
            @pl.loop(0, H, step=LANES)
            def _(c):
                zbuf[r, pl.ds(c, LANES)] = z16

        r_base = sid * ROWS_PS

        @pl.loop(0, ROWS_PS, step=ZB)
        def _(r0):
            pltpu.sync_copy(zbuf, acc.at[pl.ds(r_base + r0, ZB)])

        @pl.when(sid == 0)
        def _():
            pltpu.sync_copy(zbuf.at[pl.ds(0, REM)],
                            acc.at[pl.ds(NS * ROWS_PS, REM)])

        base = wid * EPW

        def start_gather(j, b):
            pltpu.async_copy(g_hbm.at[src_v.at[pl.ds(j * K, K)]], rb[b],
                             gs[b])

        def wait_gather(b):
            pltpu.make_async_copy(g_hbm.at[src_v.at[pl.ds(0, K)]], rb[b],
                                  gs[b]).wait()

        def start_dst(j, b):
            pltpu.async_copy(dst_hbm.at[pl.ds(base + j * K, K)], db[b],
                             ds[b])

        def wait_dst(b):
            pltpu.make_async_copy(dst_hbm.at[pl.ds(base, K)], db[b],
                                  ds[b]).wait()

        def start_scatter(b):
            pltpu.async_copy(rb[b], acc.at[db[b]], ss[b], add=True)

        def wait_scatter(b):
            pltpu.make_async_copy(rb[b], acc.at[db[b]], ss[b]).wait()

        for b in range(NB - 1):
            start_dst(b, b)
        pltpu.sync_copy(src_hbm.at[pl.ds(base, EPW)], src_v)
        for b in range(NB - 1):
            start_gather(b, b)

        plsc.subcore_barrier()

        @pl.loop(0, NCHUNK, step=NB)
        def _(jj):
            for b in range(NB):
                pb = (b + NB - 1) % NB
                wait_gather(b)
                wait_dst(b)
                start_scatter(b)
                if b == 0:
                    @pl.when(jj > 0)
                    def _():
                        wait_scatter(pb)
                    start_dst(jj + NB - 1, pb)
                    start_gather(jj + NB - 1, pb)
                else:
                    wait_scatter(pb)

                    @pl.when(jj + b + NB - 1 < NCHUNK)
                    def _(b=b, pb=pb, jj=jj):
                        start_dst(jj + b + NB - 1, pb)
                        start_gather(jj + b + NB - 1, pb)

        wait_scatter(NB - 1)

        plsc.subcore_barrier()
        pltpu.sync_copy(acc.at[pl.ds(r_base, ROWS_PS)],
                        out_hbm.at[cid, pl.ds(r_base, ROWS_PS)])

        @pl.when(sid == 0)
        def _():
            pltpu.sync_copy(acc.at[pl.ds(NS * ROWS_PS, REM)],
                            out_hbm.at[cid, pl.ds(NS * ROWS_PS, REM)])

    return scat_kernel


def _dinv_prep(deg_p):
    N = deg_p.shape[1]

    def body(dp_ref, out_ref):
        s = jnp.sum(dp_ref[...], axis=0, keepdims=True) + 1.0
        out_ref[...] = lax.rsqrt(s).T

    return pl.pallas_call(
        body,
        out_shape=jax.ShapeDtypeStruct((N, 1), F32),
    )(deg_p)


def _stage_in(x, W_in, b_in, Wc0, dinv, R=1000):
    N, D = x.shape
    H = W_in.shape[1]
    assert N % R == 0

    def body(x_ref, win_ref, bin_ref, wc_ref, dinv_ref, g_ref):
        h0 = jnp.dot(x_ref[...], win_ref[...],
                     preferred_element_type=F32) + bin_ref[...]
        g_ref[...] = jnp.dot(h0, wc_ref[...],
                             preferred_element_type=F32) * dinv_ref[...]

    return pl.pallas_call(
        body,
        grid=(N // R,),
        in_specs=[
            pl.BlockSpec((R, D), lambda i: (i, 0)),
            pl.BlockSpec((D, H), lambda i: (0, 0)),
            pl.BlockSpec((1, H), lambda i: (0, 0)),
            pl.BlockSpec((H, H), lambda i: (0, 0)),
            pl.BlockSpec((R, 1), lambda i: (i, 0)),
        ],
        out_specs=pl.BlockSpec((R, H), lambda i: (i, 0)),
        out_shape=jax.ShapeDtypeStruct((N, H), F32),
    )(x, W_in, b_in.reshape(1, H), Wc0, dinv)


def _post_conv(p, g, dinv, bc, lng, lnb):
    s = (p[0] + p[1] + g) * dinv + bc
    m = jnp.mean(s, axis=-1, keepdims=True)
    d = s - m
    v = jnp.mean(d * d, axis=-1, keepdims=True)
    t = d * lax.rsqrt(v + 1e-5) * lng + lnb
    return jnp.maximum(t, 0.0)


def _stage_mid(part, g, hprev, dinv, bc_i, lng_i, lnb_i, Wnext, residual,
               R=1000):
    N, H = g.shape
    assert N % R == 0

    if residual:
        def body(part_ref, g_ref, hp_ref, dinv_ref, bc_ref, lng_ref, lnb_ref,
                 wn_ref, h_ref, gn_ref):
            h = _post_conv(part_ref[...], g_ref[...], dinv_ref[...],
                           bc_ref[...], lng_ref[...], lnb_ref[...])
            h = h + hp_ref[...]
            h_ref[...] = h
            gn_ref[...] = jnp.dot(h, wn_ref[...],
                                  preferred_element_type=F32) * dinv_ref[...]
        extra = [hprev]
        extra_specs = [pl.BlockSpec((R, H), lambda i: (i, 0))]
    else:
        def body(part_ref, g_ref, dinv_ref, bc_ref, lng_ref, lnb_ref,
                 wn_ref, h_ref, gn_ref):
            h = _post_conv(part_ref[...], g_ref[...], dinv_ref[...],
                           bc_ref[...], lng_ref[...], lnb_ref[...])
            h_ref[...] = h
            gn_ref[...] = jnp.dot(h, wn_ref[...],
                                  preferred_element_type=F32) * dinv_ref[...]
        extra = []
        extra_specs = []

    return pl.pallas_call(
        body,
        grid=(N // R,),
        in_specs=[
            pl.BlockSpec((2, R, H), lambda i: (0, i, 0)),
            pl.BlockSpec((R, H), lambda i: (i, 0)),
            *extra_specs,
            pl.BlockSpec((R, 1), lambda i: (i, 0)),
            pl.BlockSpec((1, H), lambda i: (0, 0)),
            pl.BlockSpec((1, H), lambda i: (0, 0)),
            pl.BlockSpec((1, H), lambda i: (0, 0)),
            pl.BlockSpec((H, H), lambda i: (0, 0)),
        ],
        out_specs=[
            pl.BlockSpec((R, H), lambda i: (i, 0)),
            pl.BlockSpec((R, H), lambda i: (i, 0)),
        ],
        out_shape=[
            jax.ShapeDtypeStruct((N, H), F32),
            jax.ShapeDtypeStruct((N, H), F32),
        ],
    )(part, g, *extra, dinv, bc_i.reshape(1, H), lng_i.reshape(1, H),
      lnb_i.reshape(1, H), Wnext)


def _stage_last(part, g, hprev, dinv, bc_i, lng_i, lnb_i, R=1000):
    N, H = g.shape
    assert N % R == 0

    def body(part_ref, g_ref, hp_ref, dinv_ref, bc_ref, lng_ref, lnb_ref,
             out_ref):
        h = _post_conv(part_ref[...], g_ref[...], dinv_ref[...],
                       bc_ref[...], lng_ref[...], lnb_ref[...])
        h = h + hp_ref[...]

        @pl.when(pl.program_id(0) == 0)
        def _():
            out_ref[...] = jnp.zeros_like(out_ref)

        out_ref[...] += jnp.sum(h, axis=0, keepdims=True)

    return pl.pallas_call(
        body,
        grid=(N // R,),
        in_specs=[
            pl.BlockSpec((2, R, H), lambda i: (0, i, 0)),
            pl.BlockSpec((R, H), lambda i: (i, 0)),
            pl.BlockSpec((R, H), lambda i: (i, 0)),
            pl.BlockSpec((R, 1), lambda i: (i, 0)),
            pl.BlockSpec((1, H), lambda i: (0, 0)),
            pl.BlockSpec((1, H), lambda i: (0, 0)),
            pl.BlockSpec((1, H), lambda i: (0, 0)),
        ],
        out_specs=pl.BlockSpec((1, H), lambda i: (0, 0)),
        out_shape=jax.ShapeDtypeStruct((1, H), F32),
    )(part, g, hprev, dinv, bc_i.reshape(1, H), lng_i.reshape(1, H),
      lnb_i.reshape(1, H))


def _last_and_head(part, g, hprev, dinv, bc_i, lng_i, lnb_i, prev_sums,
                   Wih_t, Whh_t, bih, bhh, W1, b1, W2r, b2, R=1000):
    N, H = g.shape
    assert N % R == 0
    NG = N // R
    T = prev_sums.shape[0] + 1
    Hmid = W1.shape[1]

    def body(part_ref, g_ref, hp_ref, dinv_ref, bc_ref, lng_ref, lnb_ref,
             prev_ref, wih_ref, whh_ref, bih_ref, bhh_ref, w1_ref, b1_ref,
             w2_ref, b2_ref, pred_ref, final_ref, acc_ref):
        h3 = _post_conv(part_ref[...], g_ref[...], dinv_ref[...],
                        bc_ref[...], lng_ref[...], lnb_ref[...])
        h3 = h3 + hp_ref[...]

        @pl.when(pl.program_id(0) == 0)
        def _():
            acc_ref[...] = jnp.zeros_like(acc_ref)

        acc_ref[...] += jnp.sum(h3, axis=0, keepdims=True)

        @pl.when(pl.program_id(0) == NG - 1)
        def _():
            _head_compute(prev_ref, acc_ref, wih_ref, whh_ref, bih_ref,
                          bhh_ref, w1_ref, b1_ref, w2_ref, b2_ref,
                          pred_ref, final_ref, T, H, N)

    return pl.pallas_call(
        body,
        grid=(NG,),
        in_specs=[
            pl.BlockSpec((2, R, H), lambda i: (0, i, 0)),
            pl.BlockSpec((R, H), lambda i: (i, 0)),
            pl.BlockSpec((R, H), lambda i: (i, 0)),
            pl.BlockSpec((R, 1), lambda i: (i, 0)),
            pl.BlockSpec((1, H), lambda i: (0, 0)),
            pl.BlockSpec((1, H), lambda i: (0, 0)),
            pl.BlockSpec((1, H), lambda i: (0, 0)),
            pl.BlockSpec(prev_sums.shape, lambda i: (0, 0)),
            pl.BlockSpec(Wih_t.shape, lambda i: (0, 0, 0)),
            pl.BlockSpec(Whh_t.shape, lambda i: (0, 0, 0)),
            pl.BlockSpec(bih.shape, lambda i: (0, 0)),
            pl.BlockSpec(bhh.shape, lambda i: (0, 0)),
            pl.BlockSpec(W1.shape, lambda i: (0, 0)),
            pl.BlockSpec((1, Hmid), lambda i: (0, 0)),
            pl.BlockSpec(W2r.shape, lambda i: (0, 0)),
            pl.BlockSpec((1, 1), lambda i: (0, 0)),
        ],
        out_specs=[
            pl.BlockSpec((1, 1), lambda i: (0, 0)),
            pl.BlockSpec((1, H), lambda i: (0, 0)),
        ],
        out_shape=[
            jax.ShapeDtypeStruct((1, 1), F32),
            jax.ShapeDtypeStruct((1, H), F32),
        ],
        scratch_shapes=[pltpu.VMEM((1, H), F32)],
    )(part, g, hprev, dinv, bc_i.reshape(1, H), lng_i.reshape(1, H),
      lnb_i.reshape(1, H), prev_sums, Wih_t, Whh_t, bih, bhh, W1,
      b1.reshape(1, Hmid), W2r, b2.reshape(1, 1))


def _head_compute(prev_ref, acc_ref, wih_ref, whh_ref, bih_ref, bhh_ref,
                  w1_ref, b1_ref, w2_ref, b2_ref, pred_ref, final_ref,
                  T, H, N):
        inv_n = F32(1.0 / N)
        xs = [prev_ref[pl.ds(t, 1), :] * inv_n for t in range(T - 1)]
        xs.append(acc_ref[...] * inv_n)
        for l in range(2):
            wih = wih_ref[l]
            whh = whh_ref[l]
            b = bih_ref[pl.ds(l, 1), :] + bhh_ref[pl.ds(l, 1), :]
            h = jnp.zeros((1, H), F32)
            c = jnp.zeros((1, H), F32)
            ys = []
            for t in range(T):
                z = (jnp.dot(xs[t], wih, preferred_element_type=F32)
                     + jnp.dot(h, whh, preferred_element_type=F32) + b)
                zi = z[:, 0:H]
                zf = z[:, H:2 * H]
                zg = z[:, 2 * H:3 * H]
                zo = z[:, 3 * H:4 * H]
                c = jax.nn.sigmoid(zf) * c + jax.nn.sigmoid(zi) * jnp.tanh(zg)
                h = jax.nn.sigmoid(zo) * jnp.tanh(c)
                ys.append(h)
            xs = ys
        final = xs[-1]
        hmid = jnp.maximum(
            jnp.dot(final, w1_ref[...], preferred_element_type=F32)
            + b1_ref[...], 0.0)
        pred_ref[...] = (jnp.sum(hmid * w2_ref[...], axis=-1, keepdims=True)
                         + b2_ref[...])
        final_ref[...] = final


@jax.jit
def kernel(x_0, x_1, x_2, x_3,
           edge_index_0, edge_index_1, edge_index_2, edge_index_3,
           W_in, b_in, Wc, bc, ln_g, ln_b,
           lstm_Wih, lstm_Whh, lstm_bih, lstm_bhh, W1, b1, W2, b2):
    xs = [x_0, x_1, x_2, x_3]
    eis = [edge_index_0, edge_index_1, edge_index_2, edge_index_3]
    N, D = x_0.shape
    E = edge_index_0.shape[1]
    H = W_in.shape[1]
    T = len(xs)

    deg_kernel = _make_deg_kernel(N, E, T)
    scat_kernel = _make_scatter_kernel(N, E, H)

    deg_ps = deg_kernel(*[ei[1] for ei in eis])

    emb_sums = []
    for t in range(T):
        src = eis[t][0]
        dst = eis[t][1]
        dinv = _dinv_prep(deg_ps[t].reshape(-1, N))
        g0 = _stage_in(xs[t], W_in, b_in, Wc[0], dinv)
        part0 = scat_kernel(g0, src, dst)
        h1, g1 = _stage_mid(part0, g0, None, dinv, bc[0], ln_g[0], ln_b[0],
                            Wc[1], residual=False)
        part1 = scat_kernel(g1, src, dst)
        h2, g2 = _stage_mid(part1, g1, h1, dinv, bc[1], ln_g[1], ln_b[1],
                            Wc[2], residual=True)
        part2 = scat_kernel(g2, src, dst)
        if t < T - 1:
            emb_sums.append(_stage_last(part2, g2, h2, dinv, bc[2], ln_g[2],
                                        ln_b[2]))
        else:
            last_args = (part2, g2, h2, dinv)

    prev_sums = jnp.concatenate(emb_sums, axis=0)
    Wih_t = jnp.swapaxes(lstm_Wih, 1, 2)
    Whh_t = jnp.swapaxes(lstm_Whh, 1, 2)
    pred, final = _last_and_head(*last_args, bc[2], ln_g[2], ln_b[2],
                                 prev_sums, Wih_t, Whh_t, lstm_bih,
                                 lstm_bhh, W1, b1, W2.reshape(1, -1), b2)
    return pred, final
